# Initial kernel scaffold; baseline (speedup 1.0000x reference)
import jax
import jax.numpy as jnp
from jax.experimental import pallas as pl


def kernel(feature_maps, boxes):
    raise NotImplementedError("write your pallas kernel here")


# speedup vs baseline: 62.3478x; 62.3478x over previous
import jax
import jax.numpy as jnp
from jax.experimental import pallas as pl
from jax.experimental.pallas import tpu as pltpu

_OUT = 7
_SR = 2
_S = _OUT * _SR
_CHUNK = 32
_ROWS = _CHUNK * _OUT * _OUT


def _roi_body(p_ref, f_ref, o_ref):
    p = p_ref[0]
    ys_a = p[:, 0:1]
    ys_b = p[:, 1:2]
    xs_a = p[:, 2:3]
    xs_b = p[:, 3:4]
    li = jax.lax.broadcasted_iota(jnp.int32, (1, 1024), 1)
    hh = (li // 32).astype(jnp.float32)
    ww = (li % 32).astype(jnp.float32)

    def tri(d):
        return jnp.maximum(1.0 - jnp.abs(d), 0.0)

    ay = tri(ys_a - hh) + tri(ys_b - hh)
    ax = tri(xs_a - ww) + tri(xs_b - ww)
    wm = ay * ax
    f = f_ref[0] * 0.25
    o_ref[0] = jnp.dot(wm, f, preferred_element_type=jnp.float32)


def _sample_coords(boxes, H, W):
    N = boxes.shape[0]
    scale = jnp.array([W - 1.0, H - 1.0, W - 1.0, H - 1.0], dtype=boxes.dtype)
    b = boxes / scale
    x1 = jnp.maximum(b[:, 0], 0.0)
    y1 = jnp.maximum(b[:, 1], 0.0)
    x2 = jnp.minimum(b[:, 2], 1.0)
    y2 = jnp.minimum(b[:, 3], 1.0)
    bin_h = (y2 - y1) / _OUT
    bin_w = (x2 - x1) / _OUT
    gy1 = y1 + 0.5 * bin_h / _SR
    gx1 = x1 + 0.5 * bin_w / _SR
    gy2 = y2 - 0.5 * bin_h / _SR
    gx2 = x2 - 0.5 * bin_w / _SR
    i = jnp.arange(_S, dtype=boxes.dtype)
    ys = gy1[:, None] * (H - 1) + i[None, :] * ((gy2 - gy1) * (H - 1) / (_S - 1))[:, None]
    xs = gx1[:, None] * (W - 1) + i[None, :] * ((gx2 - gx1) * (W - 1) / (_S - 1))[:, None]
    ys_a, ys_b = ys[:, 0::2], ys[:, 1::2]
    xs_a, xs_b = xs[:, 0::2], xs[:, 1::2]
    ys_a = jnp.broadcast_to(ys_a[:, :, None], (N, 7, 7)).reshape(N, 49)
    ys_b = jnp.broadcast_to(ys_b[:, :, None], (N, 7, 7)).reshape(N, 49)
    xs_a = jnp.broadcast_to(xs_a[:, None, :], (N, 7, 7)).reshape(N, 49)
    xs_b = jnp.broadcast_to(xs_b[:, None, :], (N, 7, 7)).reshape(N, 49)
    return jnp.stack([ys_a, ys_b, xs_a, xs_b], axis=-1)


def kernel(feature_maps, boxes):
    B, H, W, C = feature_maps.shape
    R = boxes.shape[1]
    n_chunks = R // _CHUNK
    params = _sample_coords(boxes.reshape(B * R, 4), H, W)
    params = params.reshape(B, R * 49, 4)
    fmap = feature_maps.reshape(B, H * W, C)
    out = pl.pallas_call(
        _roi_body,
        grid=(B, n_chunks),
        in_specs=[
            pl.BlockSpec((1, _ROWS, 4), lambda b, c: (b, c, 0)),
            pl.BlockSpec((1, H * W, C), lambda b, c: (b, 0, 0)),
        ],
        out_specs=pl.BlockSpec((1, _ROWS, C), lambda b, c: (b, c, 0)),
        out_shape=jax.ShapeDtypeStruct((B, R * 49, C), jnp.float32),
        compiler_params=pltpu.CompilerParams(
            dimension_semantics=("parallel", "arbitrary"),
        ),
    )(params, fmap)
    return out.reshape(B, R, _OUT, _OUT, C)


# speedup vs baseline: 161.3382x; 2.5877x over previous
import jax
import jax.numpy as jnp
from jax.experimental import pallas as pl
from jax.experimental.pallas import tpu as pltpu

_OUT = 7
_SR = 2
_S = _OUT * _SR
_CHUNK = 32
_RY = _CHUNK * _OUT
_RX = _CHUNK * 8
_ROWS = _CHUNK * _OUT * 8


def _roi_body(py_ref, px_ref, f_ref, o_ref):
    py = py_ref[0]
    px = px_ref[0]
    li = jax.lax.broadcasted_iota(jnp.int32, (1, 1024), 1)
    hh = (li // 32).astype(jnp.float32)
    ww = (li % 32).astype(jnp.float32)

    def tri(d):
        return jnp.maximum(1.0 - jnp.abs(d), 0.0)

    ay = tri(py[:, 0:1] - hh) + tri(py[:, 1:2] - hh)
    ax = tri(px[:, 0:1] - ww) + tri(px[:, 1:2] - ww)
    ayb = jnp.broadcast_to(
        ay.reshape(_RY, 1, 1024), (_RY, 8, 1024)).reshape(_ROWS, 1024)
    axb = jnp.broadcast_to(
        ax.reshape(_CHUNK, 1, 8, 1024), (_CHUNK, _OUT, 8, 1024)
    ).reshape(_ROWS, 1024)
    wm = ayb * axb
    f = f_ref[0].reshape(1024, 256) * 0.25
    res = jnp.dot(wm, f, preferred_element_type=jnp.float32)
    o_ref[0] = res.reshape(_CHUNK, _OUT, 8, 256)[:, :, :_OUT, :]


def _sample_coords(boxes, H, W):
    N = boxes.shape[0]
    scale = jnp.array([W - 1.0, H - 1.0, W - 1.0, H - 1.0], dtype=boxes.dtype)
    b = boxes / scale
    x1 = jnp.maximum(b[:, 0], 0.0)
    y1 = jnp.maximum(b[:, 1], 0.0)
    x2 = jnp.minimum(b[:, 2], 1.0)
    y2 = jnp.minimum(b[:, 3], 1.0)
    bin_h = (y2 - y1) / _OUT
    bin_w = (x2 - x1) / _OUT
    gy1 = y1 + 0.5 * bin_h / _SR
    gx1 = x1 + 0.5 * bin_w / _SR
    gy2 = y2 - 0.5 * bin_h / _SR
    gx2 = x2 - 0.5 * bin_w / _SR
    i = jnp.arange(_S, dtype=boxes.dtype)
    ys = gy1[:, None] * (H - 1) + i[None, :] * ((gy2 - gy1) * (H - 1) / (_S - 1))[:, None]
    xs = gx1[:, None] * (W - 1) + i[None, :] * ((gx2 - gx1) * (W - 1) / (_S - 1))[:, None]
    py = jnp.stack([ys[:, 0::2], ys[:, 1::2]], axis=-1)
    px = jnp.stack([xs[:, 0::2], xs[:, 1::2]], axis=-1)
    pad = jnp.full((N, 1, 2), -100.0, dtype=boxes.dtype)
    px = jnp.concatenate([px, pad], axis=1)
    return py, px


def kernel(feature_maps, boxes):
    B, H, W, C = feature_maps.shape
    R = boxes.shape[1]
    n_chunks = R // _CHUNK
    py, px = _sample_coords(boxes.reshape(B * R, 4), H, W)
    py = py.reshape(B, R * _OUT, 2)
    px = px.reshape(B, R * 8, 2)
    return pl.pallas_call(
        _roi_body,
        grid=(B, n_chunks),
        in_specs=[
            pl.BlockSpec((1, _RY, 2), lambda b, c: (b, c, 0)),
            pl.BlockSpec((1, _RX, 2), lambda b, c: (b, c, 0)),
            pl.BlockSpec((1, H, W, C), lambda b, c: (b, 0, 0, 0)),
        ],
        out_specs=pl.BlockSpec(
            (1, _CHUNK, _OUT, _OUT, C), lambda b, c: (b, c, 0, 0, 0)),
        out_shape=jax.ShapeDtypeStruct((B, R, _OUT, _OUT, C), jnp.float32),
        compiler_params=pltpu.CompilerParams(
            dimension_semantics=("parallel", "arbitrary"),
        ),
    )(py, px, feature_maps)


# speedup vs baseline: 162.4270x; 1.0067x over previous
import jax
import jax.numpy as jnp
from jax.experimental import pallas as pl
from jax.experimental.pallas import tpu as pltpu

_OUT = 7
_SR = 2
_S = _OUT * _SR
_CHUNK = 32
_RY = _CHUNK * _OUT
_RX = _CHUNK * 8
_ROWS = _CHUNK * _OUT * 8


def _roi_body(py_ref, px_ref, f_ref, o_ref):
    py = py_ref[0]
    px = px_ref[0]
    li = jax.lax.broadcasted_iota(jnp.int32, (1, 1024), 1)
    hh = (li // 32).astype(jnp.float32)
    ww = (li % 32).astype(jnp.float32)

    def tri(d):
        return jnp.maximum(1.0 - jnp.abs(d), 0.0)

    ay = tri(py[:, 0:1] - hh) + tri(py[:, 1:2] - hh)
    ax = tri(px[:, 0:1] - ww) + tri(px[:, 1:2] - ww)
    ayb = jnp.broadcast_to(
        ay.reshape(_RY, 1, 1024), (_RY, 8, 1024)).reshape(_ROWS, 1024)
    axb = jnp.broadcast_to(
        ax.reshape(_CHUNK, 1, 8, 1024), (_CHUNK, _OUT, 8, 1024)
    ).reshape(_ROWS, 1024)
    wm = ayb * axb
    f = f_ref[0].reshape(1024, 256) * 0.25
    res = jnp.dot(wm, f, preferred_element_type=jnp.float32)
    o_ref[0] = res.reshape(_CHUNK, _OUT, 8, 256)


def _sample_coords(boxes, H, W):
    N = boxes.shape[0]
    scale = jnp.array([W - 1.0, H - 1.0, W - 1.0, H - 1.0], dtype=boxes.dtype)
    b = boxes / scale
    x1 = jnp.maximum(b[:, 0], 0.0)
    y1 = jnp.maximum(b[:, 1], 0.0)
    x2 = jnp.minimum(b[:, 2], 1.0)
    y2 = jnp.minimum(b[:, 3], 1.0)
    bin_h = (y2 - y1) / _OUT
    bin_w = (x2 - x1) / _OUT
    gy1 = y1 + 0.5 * bin_h / _SR
    gx1 = x1 + 0.5 * bin_w / _SR
    gy2 = y2 - 0.5 * bin_h / _SR
    gx2 = x2 - 0.5 * bin_w / _SR
    i = jnp.arange(_S, dtype=boxes.dtype)
    ys = gy1[:, None] * (H - 1) + i[None, :] * ((gy2 - gy1) * (H - 1) / (_S - 1))[:, None]
    xs = gx1[:, None] * (W - 1) + i[None, :] * ((gx2 - gx1) * (W - 1) / (_S - 1))[:, None]
    py = jnp.stack([ys[:, 0::2], ys[:, 1::2]], axis=-1)
    px = jnp.stack([xs[:, 0::2], xs[:, 1::2]], axis=-1)
    pad = jnp.full((N, 1, 2), -100.0, dtype=boxes.dtype)
    px = jnp.concatenate([px, pad], axis=1)
    return py, px


def kernel(feature_maps, boxes):
    B, H, W, C = feature_maps.shape
    R = boxes.shape[1]
    n_chunks = R // _CHUNK
    py, px = _sample_coords(boxes.reshape(B * R, 4), H, W)
    py = py.reshape(B, R * _OUT, 2)
    px = px.reshape(B, R * 8, 2)
    return pl.pallas_call(
        _roi_body,
        grid=(B, n_chunks),
        in_specs=[
            pl.BlockSpec((1, _RY, 2), lambda b, c: (b, c, 0)),
            pl.BlockSpec((1, _RX, 2), lambda b, c: (b, c, 0)),
            pl.BlockSpec((1, H, W, C), lambda b, c: (b, 0, 0, 0)),
        ],
        out_specs=pl.BlockSpec(
            (1, _CHUNK, _OUT, 8, C), lambda b, c: (b, c, 0, 0, 0)),
        out_shape=jax.ShapeDtypeStruct((B, R, _OUT, _OUT, C), jnp.float32),
        compiler_params=pltpu.CompilerParams(
            dimension_semantics=("parallel", "arbitrary"),
        ),
    )(py, px, feature_maps)


# speedup vs baseline: 165.2673x; 1.0175x over previous
import jax
import jax.numpy as jnp
from jax.experimental import pallas as pl
from jax.experimental.pallas import tpu as pltpu

_OUT = 7
_SR = 2
_S = _OUT * _SR
_CHUNK = 64
_RY = _CHUNK * _OUT
_RX = _CHUNK * 8
_ROWS = _CHUNK * _OUT * 8


def _roi_body(py_ref, px_ref, f_ref, o_ref):
    py = py_ref[0]
    px = px_ref[0]
    li = jax.lax.broadcasted_iota(jnp.int32, (1, 1024), 1)
    hh = (li // 32).astype(jnp.float32)
    ww = (li % 32).astype(jnp.float32)

    def tri(d):
        return jnp.maximum(1.0 - jnp.abs(d), 0.0)

    ay = tri(py[:, 0:1] - hh) + tri(py[:, 1:2] - hh)
    ax = tri(px[:, 0:1] - ww) + tri(px[:, 1:2] - ww)
    ayb = jnp.broadcast_to(
        ay.reshape(_RY, 1, 1024), (_RY, 8, 1024)).reshape(_ROWS, 1024)
    axb = jnp.broadcast_to(
        ax.reshape(_CHUNK, 1, 8, 1024), (_CHUNK, _OUT, 8, 1024)
    ).reshape(_ROWS, 1024)
    wm = ayb * axb
    f = f_ref[0].reshape(1024, 256) * 0.25
    res = jnp.dot(wm, f, preferred_element_type=jnp.float32)
    o_ref[0] = res.reshape(_CHUNK, _OUT, 8, 256)


def _sample_coords(boxes, H, W):
    N = boxes.shape[0]
    scale = jnp.array([W - 1.0, H - 1.0, W - 1.0, H - 1.0], dtype=boxes.dtype)
    b = boxes / scale
    x1 = jnp.maximum(b[:, 0], 0.0)
    y1 = jnp.maximum(b[:, 1], 0.0)
    x2 = jnp.minimum(b[:, 2], 1.0)
    y2 = jnp.minimum(b[:, 3], 1.0)
    bin_h = (y2 - y1) / _OUT
    bin_w = (x2 - x1) / _OUT
    gy1 = y1 + 0.5 * bin_h / _SR
    gx1 = x1 + 0.5 * bin_w / _SR
    gy2 = y2 - 0.5 * bin_h / _SR
    gx2 = x2 - 0.5 * bin_w / _SR
    i = jnp.arange(_S, dtype=boxes.dtype)
    ys = gy1[:, None] * (H - 1) + i[None, :] * ((gy2 - gy1) * (H - 1) / (_S - 1))[:, None]
    xs = gx1[:, None] * (W - 1) + i[None, :] * ((gx2 - gx1) * (W - 1) / (_S - 1))[:, None]
    py = jnp.stack([ys[:, 0::2], ys[:, 1::2]], axis=-1)
    px = jnp.stack([xs[:, 0::2], xs[:, 1::2]], axis=-1)
    pad = jnp.full((N, 1, 2), -100.0, dtype=boxes.dtype)
    px = jnp.concatenate([px, pad], axis=1)
    return py, px


def kernel(feature_maps, boxes):
    B, H, W, C = feature_maps.shape
    R = boxes.shape[1]
    n_chunks = R // _CHUNK
    py, px = _sample_coords(boxes.reshape(B * R, 4), H, W)
    py = py.reshape(B, R * _OUT, 2)
    px = px.reshape(B, R * 8, 2)
    return pl.pallas_call(
        _roi_body,
        grid=(B, n_chunks),
        in_specs=[
            pl.BlockSpec((1, _RY, 2), lambda b, c: (b, c, 0)),
            pl.BlockSpec((1, _RX, 2), lambda b, c: (b, c, 0)),
            pl.BlockSpec((1, H, W, C), lambda b, c: (b, 0, 0, 0)),
        ],
        out_specs=pl.BlockSpec(
            (1, _CHUNK, _OUT, 8, C), lambda b, c: (b, c, 0, 0, 0)),
        out_shape=jax.ShapeDtypeStruct((B, R, _OUT, _OUT, C), jnp.float32),
        compiler_params=pltpu.CompilerParams(
            dimension_semantics=("parallel", "arbitrary"),
        ),
    )(py, px, feature_maps)
